# baseline (speedup 1.0000x reference)
import jax
import jax.numpy as jnp
from jax import lax
from jax.experimental import pallas as pl
from jax.experimental.pallas import tpu as pltpu
from jax.experimental.pallas import tpu_sc as plsc

NW_SPIXELS = 32
NH_SPIXELS = 32
K = NW_SPIXELS * NH_SPIXELS
LANES = 16
NUM_CORES = 2
NUM_SUBCORES = 16
NUM_WORKERS = NUM_CORES * NUM_SUBCORES

CHUNK = 2048


def _sc_calc_assoc(pf_hbm, sf_hbm, im_hbm, out_hbm, ptab_v, idx_v,
                   pix_v, out_v, isem0, isem1, psem0, psem1, osem):
  B, C, HW = pf_hbm.shape
  workers_per_batch = NUM_WORKERS // B
  per_worker = HW // workers_per_batch
  n_chunks = per_worker // CHUNK

  isems = (isem0, isem1)
  psems = (psem0, psem1)

  wid = lax.axis_index("s") * NUM_CORES + lax.axis_index("c")
  b = wid // workers_per_batch
  base = (wid % workers_per_batch) * per_worker

  pltpu.sync_copy(sf_hbm.at[b], pix_v.at[0, :, pl.ds(0, K)])

  def pack_body(i, carry):
    s = pl.multiple_of(i * LANES, LANES)
    for cp in range(C // 2):
      a = pix_v[0, 2 * cp, pl.ds(s, LANES)]
      bb = pix_v[0, 2 * cp + 1, pl.ds(s, LANES)]
      packed = plsc.pack(a, bb, format=plsc.PackFormat.INTERLEAVED)
      ptab_v[cp, pl.ds(s, LANES)] = plsc.bitcast(packed, jnp.int32)
    return carry

  lax.fori_loop(0, K // LANES, pack_body, 0)

  def in_copies(ci, k):
    off = base + ci * CHUNK
    return (
        pltpu.make_async_copy(im_hbm.at[b, pl.ds(off, CHUNK)], idx_v.at[k],
                              isems[k]),
        pltpu.make_async_copy(pf_hbm.at[b, :, pl.ds(off, CHUNK)], pix_v.at[k],
                              psems[k]),
    )

  def out_copy(ci):
    off = base + ci * CHUNK
    return pltpu.make_async_copy(out_v,
                                 out_hbm.at[b, :, pl.ds(off, CHUNK)], osem)

  def start_in(ci, k):
    for cp in in_copies(ci, k):
      cp.start()

  start_in(0, 0)

  def compute(ci, k):
    idx_b, pix_b, out_b = idx_v.at[k], pix_v.at[k], out_v

    def vec_body(i, carry2):
      s = pl.multiple_of(i * LANES, LANES)
      idx = idx_b[pl.ds(s, LANES)]
      sx = jnp.bitwise_and(idx, NW_SPIXELS - 1)
      sy = jnp.right_shift(idx, 5)
      nys = (jnp.maximum(sy - 1, 0), sy, jnp.minimum(sy + 1, NH_SPIXELS - 1))
      nxs = (jnp.maximum(sx - 1, 0), sx, jnp.minimum(sx + 1, NW_SPIXELS - 1))
      rows = [jnp.left_shift(ny, 5) for ny in nys]
      nidx = [row + nx for row in rows for nx in nxs]
      pp = []
      for cp in range(C // 2):
        a = pix_b[2 * cp, pl.ds(s, LANES)]
        bb = pix_b[2 * cp + 1, pl.ds(s, LANES)]
        pp.append(plsc.pack(a, bb, format=plsc.PackFormat.INTERLEAVED))
      acc = [None] * 9
      for cp in range(C // 2):
        row_ref = ptab_v.at[cp]
        for n in range(9):
          g = plsc.bitcast(plsc.load_gather(row_ref, [nidx[n]]), jnp.bfloat16)
          t = pp[cp] - g
          t = t * t
          acc[n] = t if acc[n] is None else acc[n] + t
      for n in range(9):
        ua, ub = plsc.unpack(acc[n], format=plsc.PackFormat.INTERLEAVED)
        out_b[n, pl.ds(s, LANES)] = ua + ub
      return carry2

    lax.fori_loop(0, CHUNK // LANES, vec_body, 0)

  def outer(j, carry):
    ci0 = 2 * j
    for k in (0, 1):
      ci = ci0 + k

      @pl.when(ci + 1 < n_chunks)
      def _():
        start_in(ci + 1, 1 - k)

      for cp in in_copies(ci, k):
        cp.wait()

      @pl.when(ci >= 1)
      def _():
        out_copy(ci - 1).wait()

      compute(ci, k)
      out_copy(ci).start()
    return carry

  lax.fori_loop(0, n_chunks // 2, outer, 0)
  out_copy(n_chunks - 1).wait()


def kernel(pixel_feats, spixel_feats, index_map):
  B, C, H, W = pixel_feats.shape
  HW = H * W
  pf = pixel_feats.reshape(B, C, HW)
  im = index_map.reshape(B, HW)

  mesh = plsc.VectorSubcoreMesh(
      core_axis_name="c", subcore_axis_name="s",
      num_cores=NUM_CORES, num_subcores=NUM_SUBCORES)
  run = pl.kernel(
      _sc_calc_assoc,
      out_type=jax.ShapeDtypeStruct((B, 9, HW), jnp.float32),
      mesh=mesh,
      compiler_params=pltpu.CompilerParams(use_tc_tiling_on_sc=False,
                                           needs_layout_passes=False),
      scratch_types=[
          pltpu.VMEM((C // 2, K), jnp.int32),
          pltpu.VMEM((2, CHUNK), jnp.int32),
          pltpu.VMEM((2, C, CHUNK), jnp.float32),
          pltpu.VMEM((9, CHUNK), jnp.float32),
          pltpu.SemaphoreType.DMA,
          pltpu.SemaphoreType.DMA,
          pltpu.SemaphoreType.DMA,
          pltpu.SemaphoreType.DMA,
          pltpu.SemaphoreType.DMA,
      ],
  )
  out = run(pf, spixel_feats, im)
  return out.reshape(B, 9, H, W)


# speedup vs baseline: 1.0509x; 1.0509x over previous
import jax
import jax.numpy as jnp
from jax import lax
from jax.experimental import pallas as pl
from jax.experimental.pallas import tpu as pltpu
from jax.experimental.pallas import tpu_sc as plsc

NW_SPIXELS = 32
NH_SPIXELS = 32
K = NW_SPIXELS * NH_SPIXELS
LANES = 16
NUM_CORES = 2
NUM_SUBCORES = 16
NUM_WORKERS = NUM_CORES * NUM_SUBCORES

CHUNK = 1024
PAD = LANES


def _sc_calc_assoc(pf_hbm, sf_hbm, im_hbm, out_hbm, ptab_v, idx_v,
                   pix_v, out_v, isem0, isem1, psem0, psem1, osem0, osem1):
  B, C, HW = pf_hbm.shape
  workers_per_batch = NUM_WORKERS // B
  per_worker = HW // workers_per_batch
  n_chunks = per_worker // CHUNK

  isems = (isem0, isem1)
  psems = (psem0, psem1)
  osems = (osem0, osem1)

  wid = lax.axis_index("s") * NUM_CORES + lax.axis_index("c")
  b = wid // workers_per_batch
  base = (wid % workers_per_batch) * per_worker

  pltpu.sync_copy(sf_hbm.at[b], pix_v.at[0, :, pl.ds(0, K)])

  def pack_body(i, carry):
    s = pl.multiple_of(i * LANES, LANES)
    for cp in range(C // 2):
      a = pix_v[0, 2 * cp, pl.ds(s, LANES)]
      bb = pix_v[0, 2 * cp + 1, pl.ds(s, LANES)]
      packed = plsc.pack(a, bb, format=plsc.PackFormat.INTERLEAVED)
      ptab_v[cp, pl.ds(s, LANES)] = plsc.bitcast(packed, jnp.int32)
    return carry

  lax.fori_loop(0, K // LANES, pack_body, 0)

  def in_copies(ci, k):
    off = base + ci * CHUNK
    return (
        pltpu.make_async_copy(im_hbm.at[b, pl.ds(off, CHUNK)],
                              idx_v.at[k, pl.ds(0, CHUNK)], isems[k]),
        pltpu.make_async_copy(pf_hbm.at[b, :, pl.ds(off, CHUNK)],
                              pix_v.at[k, :, pl.ds(0, CHUNK)], psems[k]),
    )

  def out_copy(ci, k):
    off = base + ci * CHUNK
    return pltpu.make_async_copy(out_v.at[k],
                                 out_hbm.at[b, :, pl.ds(off, CHUNK)], osems[k])

  def start_in(ci, k):
    for cp in in_copies(ci, k):
      cp.start()

  start_in(0, 0)

  def compute(ci, k):
    idx_b, pix_b, out_b = idx_v.at[k], pix_v.at[k], out_v.at[k]

    def prefetch(s):
      idx = idx_b[pl.ds(s, LANES)]
      sx = jnp.bitwise_and(idx, NW_SPIXELS - 1)
      sy = jnp.right_shift(idx, 5)
      nys = (jnp.maximum(sy - 1, 0), sy, jnp.minimum(sy + 1, NH_SPIXELS - 1))
      nxs = (jnp.maximum(sx - 1, 0), sx, jnp.minimum(sx + 1, NW_SPIXELS - 1))
      rows = [jnp.left_shift(ny, 5) for ny in nys]
      nidx = [row + nx for row in rows for nx in nxs]
      pp = []
      for cp in range(C // 2):
        a = pix_b[2 * cp, pl.ds(s, LANES)]
        bb = pix_b[2 * cp + 1, pl.ds(s, LANES)]
        pp.append(plsc.pack(a, bb, format=plsc.PackFormat.INTERLEAVED))
      return tuple(nidx), tuple(pp)

    def vec_body(i, carry2):
      s = pl.multiple_of(i * LANES, LANES)
      nidx, pp = carry2
      nxt = prefetch(s + LANES)
      acc = [None] * 9
      for cp in range(C // 2):
        row_ref = ptab_v.at[cp]
        for n in range(9):
          g = plsc.bitcast(plsc.load_gather(row_ref, [nidx[n]]), jnp.bfloat16)
          t = pp[cp] - g
          t = t * t
          acc[n] = t if acc[n] is None else acc[n] + t
      for n in range(9):
        ua, ub = plsc.unpack(acc[n], format=plsc.PackFormat.INTERLEAVED)
        out_b[n, pl.ds(s, LANES)] = ua + ub
      return nxt

    lax.fori_loop(0, CHUNK // LANES, vec_body, prefetch(0))

  def outer(j, carry):
    ci0 = 2 * j
    for k in (0, 1):
      ci = ci0 + k

      @pl.when(ci + 1 < n_chunks)
      def _():
        start_in(ci + 1, 1 - k)

      for cp in in_copies(ci, k):
        cp.wait()

      @pl.when(ci >= 2)
      def _():
        out_copy(ci - 2, k).wait()

      compute(ci, k)
      out_copy(ci, k).start()
    return carry

  lax.fori_loop(0, n_chunks // 2, outer, 0)
  out_copy(n_chunks - 2, 0).wait()
  out_copy(n_chunks - 1, 1).wait()


def kernel(pixel_feats, spixel_feats, index_map):
  B, C, H, W = pixel_feats.shape
  HW = H * W
  pf = pixel_feats.reshape(B, C, HW)
  im = index_map.reshape(B, HW)

  mesh = plsc.VectorSubcoreMesh(
      core_axis_name="c", subcore_axis_name="s",
      num_cores=NUM_CORES, num_subcores=NUM_SUBCORES)
  run = pl.kernel(
      _sc_calc_assoc,
      out_type=jax.ShapeDtypeStruct((B, 9, HW), jnp.float32),
      mesh=mesh,
      compiler_params=pltpu.CompilerParams(use_tc_tiling_on_sc=False,
                                           needs_layout_passes=False),
      scratch_types=[
          pltpu.VMEM((C // 2, K), jnp.int32),
          pltpu.VMEM((2, CHUNK + PAD), jnp.int32),
          pltpu.VMEM((2, C, CHUNK + PAD), jnp.float32),
          pltpu.VMEM((2, 9, CHUNK), jnp.float32),
          pltpu.SemaphoreType.DMA,
          pltpu.SemaphoreType.DMA,
          pltpu.SemaphoreType.DMA,
          pltpu.SemaphoreType.DMA,
          pltpu.SemaphoreType.DMA,
          pltpu.SemaphoreType.DMA,
      ],
  )
  out = run(pf, spixel_feats, im)
  return out.reshape(B, 9, H, W)
